# baseline (speedup 1.0000x reference)
import functools

import jax
import jax.numpy as jnp
from jax import lax
from jax.experimental import pallas as pl
from jax.experimental.pallas import tpu as pltpu
from jax.experimental.pallas import tpu_sc as plsc

E = 64
K = 2
D = 768
H = 3072
T = 2048
CAP = 77
CAPP = 80
NENT = T * K
TRASH = E * CAPP
NSLOT = (E + 1) * CAPP
CH = 128
NCH = T // CH

_HI = jax.lax.Precision.HIGHEST


def _gate_body(x_ref, wg_ref, bg_ref,
               i1_ref, i2_ref, p1_ref, p2_ref, w1_ref, w2_ref,
               k1_ref, k2_ref, s1_ref, s2_ref,
               incl_s, tot_s):
    x = x_ref[...]
    logits = jnp.dot(x, wg_ref[...], preferred_element_type=jnp.float32)
    logits = logits + bg_ref[...]
    lane = lax.broadcasted_iota(jnp.int32, (T, E), 1)
    v1 = jnp.max(logits, axis=1, keepdims=True)
    i1 = jnp.min(jnp.where(logits == v1, lane, E), axis=1, keepdims=True)
    masked = jnp.where(lane == i1, -1e30, logits)
    v2 = jnp.max(masked, axis=1, keepdims=True)
    i2 = jnp.min(jnp.where(masked == v2, lane, E), axis=1, keepdims=True)
    g1 = 1.0 / (1.0 + jnp.exp(v2 - v1))
    g2 = 1.0 - g1

    oh1 = (lane == i1).astype(jnp.float32)
    oh2 = (lane == i2).astype(jnp.float32)
    ohc = oh1 + oh2

    r = lax.broadcasted_iota(jnp.int32, (CH, CH), 0)
    c = lax.broadcasted_iota(jnp.int32, (CH, CH), 1)
    tincl = (r >= c).astype(jnp.float32)
    for cc in range(NCH):
        blk = ohc[cc * CH:(cc + 1) * CH, :]
        inc = lax.dot(tincl, blk, precision=_HI)
        incl_s[cc * CH:(cc + 1) * CH, :] = inc
        tot_s[cc:cc + 1, :] = inc[CH - 1:CH, :]
    r16 = lax.broadcasted_iota(jnp.int32, (NCH, NCH), 0)
    c16 = lax.broadcasted_iota(jnp.int32, (NCH, NCH), 1)
    tstrict = (r16 > c16).astype(jnp.float32)
    offs = lax.dot(tstrict, tot_s[...], precision=_HI)

    i1_ref[...] = i1
    i2_ref[...] = i2
    for cc in range(NCH):
        sl = slice(cc * CH, (cc + 1) * CH)
        oh1c = oh1[sl, :]
        oh2c = oh2[sl, :]
        excl = incl_s[sl, :] - (oh1c + oh2c) + offs[cc:cc + 1, :]
        p1f = jnp.sum(oh1c * excl, axis=1, keepdims=True)
        p2f = jnp.sum(oh2c * (excl + oh1c), axis=1, keepdims=True)
        p1 = (p1f + 0.5).astype(jnp.int32)
        p2 = (p2f + 0.5).astype(jnp.int32)
        keep1 = p1 < CAP
        keep2 = p2 < CAP
        tglob = cc * CH + lax.broadcasted_iota(jnp.int32, (CH, 1), 0)
        went = tglob // CH
        i1c = i1[sl, :]
        i2c = i2[sl, :]
        p1_ref[sl, :] = jnp.minimum(p1, CAPP - 1)
        p2_ref[sl, :] = jnp.minimum(p2, CAPP - 1)
        w1_ref[sl, :] = g1[sl, :] * keep1.astype(jnp.float32)
        w2_ref[sl, :] = g2[sl, :] * keep2.astype(jnp.float32)
        k1_ref[sl, :] = keep1.astype(jnp.float32)
        k2_ref[sl, :] = keep2.astype(jnp.float32)
        s1_ref[sl, :] = jnp.where(keep1, i1c * CAPP + p1, TRASH + went)
        s2_ref[sl, :] = jnp.where(keep2, i2c * CAPP + p2, TRASH + went)


def _gate(xt, Wg, bg2):
    i32 = jnp.int32
    f32 = jnp.float32
    outs = [
        jax.ShapeDtypeStruct((T, 1), i32),
        jax.ShapeDtypeStruct((T, 1), i32),
        jax.ShapeDtypeStruct((T, 1), i32),
        jax.ShapeDtypeStruct((T, 1), i32),
        jax.ShapeDtypeStruct((T, 1), f32),
        jax.ShapeDtypeStruct((T, 1), f32),
        jax.ShapeDtypeStruct((T, 1), f32),
        jax.ShapeDtypeStruct((T, 1), f32),
        jax.ShapeDtypeStruct((T, 1), i32),
        jax.ShapeDtypeStruct((T, 1), i32),
    ]
    return pl.pallas_call(
        _gate_body,
        out_shape=outs,
        scratch_shapes=[
            pltpu.VMEM((T, E), f32),
            pltpu.VMEM((NCH, E), f32),
        ],
    )(xt, Wg, bg2)


_NC = 2
_NS = 16
_NW = _NC * _NS
_EPW = NENT // _NW


_TPW = T // _NS


def _dispatch_body(x_hbm, s1_hbm, s2_hbm, buf_hbm,
                   s1_v, s2_v, rows_v, sem, sem2):
    wid = lax.axis_index("s")
    base = wid * _TPW
    cp = pltpu.async_copy(x_hbm.at[pl.ds(base, _TPW)], rows_v, sem)
    pltpu.sync_copy(s1_hbm.at[pl.ds(base, _TPW)], s1_v)
    pltpu.sync_copy(s2_hbm.at[pl.ds(base, _TPW)], s2_v)
    cp.wait()
    c1 = pltpu.async_copy(rows_v, buf_hbm.at[s1_v], sem)
    c2 = pltpu.async_copy(rows_v, buf_hbm.at[s2_v], sem2)
    c1.wait()
    c2.wait()


@functools.cache
def _dispatch_kernel():
    return pl.kernel(
        _dispatch_body,
        out_type=jax.ShapeDtypeStruct((NSLOT, D), jnp.float32),
        mesh=plsc.VectorSubcoreMesh(core_axis_name="c", subcore_axis_name="s",
                                    num_cores=1),
        scratch_types=[
            pltpu.VMEM((_TPW,), jnp.int32),
            pltpu.VMEM((_TPW,), jnp.int32),
            pltpu.VMEM((_TPW, D), jnp.float32),
            pltpu.SemaphoreType.DMA,
            pltpu.SemaphoreType.DMA,
        ],
    )


def _dispatch(xt, slot1, slot2):
    return _dispatch_kernel()(xt, slot1, slot2)


NHB = 2
HB = H // NHB


def _ffn_body(buf_ref, w1_ref, b1_ref, w2_ref, b2_ref, out_ref):
    hb = pl.program_id(1)
    xb = buf_ref[...]
    h = jnp.dot(xb, w1_ref[0], preferred_element_type=jnp.float32)
    h = jnp.maximum(h + b1_ref[0], 0.0)
    w2s = jnp.sum(w2_ref[0], axis=1, keepdims=True)
    val = jnp.dot(h, w2s, preferred_element_type=jnp.float32)

    @pl.when(hb == 0)
    def _():
        out_ref[0] = val + jnp.sum(b2_ref[...])

    @pl.when(hb != 0)
    def _():
        out_ref[0] += val

    @pl.when(hb == NHB - 1)
    def _():
        acc = out_ref[0]
        ok = (jnp.abs(acc) < 1e30) & (acc == acc)
        out_ref[0] = jnp.where(ok, acc, 0.0)


def _ffn(buf, W1, b1, W2, b2):
    return pl.pallas_call(
        _ffn_body,
        grid=(E, NHB),
        in_specs=[
            pl.BlockSpec((CAPP, D), lambda e, hb: (e, 0)),
            pl.BlockSpec((1, D, HB), lambda e, hb: (e, 0, hb)),
            pl.BlockSpec((1, 1, HB), lambda e, hb: (e, 0, hb)),
            pl.BlockSpec((1, HB, D), lambda e, hb: (e, hb, 0)),
            pl.BlockSpec((1, 1, D), lambda e, hb: (e, 0, 0)),
        ],
        out_specs=pl.BlockSpec((1, CAPP, 1), lambda e, hb: (e, 0, 0)),
        out_shape=jax.ShapeDtypeStruct((E, CAPP, 1), jnp.float32),
        compiler_params=pltpu.CompilerParams(
            dimension_semantics=("arbitrary", "arbitrary"),
        ),
    )(buf, W1, b1.reshape(E, 1, H), W2, b2.reshape(E, 1, D))


def _combine_body(ss_ref, i1_ref, i2_ref, p1_ref, p2_ref,
                  w1_ref, w2_ref, k1_ref, k2_ref, out_ref):
    ss = ss_ref[...]
    lane_e = lax.broadcasted_iota(jnp.int32, (T, E), 1)
    lane_c = lax.broadcasted_iota(jnp.int32, (T, CAPP), 1)

    oh1 = (lane_e == i1_ref[...]).astype(jnp.float32)
    a1 = lax.dot(oh1, ss, precision=_HI)
    sel1 = (lane_c == p1_ref[...]).astype(jnp.float32)
    val1 = jnp.sum(a1 * sel1, axis=1, keepdims=True)
    val1 = jnp.where(k1_ref[...] > 0.5, val1, 0.0)

    oh2 = (lane_e == i2_ref[...]).astype(jnp.float32)
    a2 = lax.dot(oh2, ss, precision=_HI)
    sel2 = (lane_c == p2_ref[...]).astype(jnp.float32)
    val2 = jnp.sum(a2 * sel2, axis=1, keepdims=True)
    val2 = jnp.where(k2_ref[...] > 0.5, val2, 0.0)

    y = w1_ref[...] * val1 + w2_ref[...] * val2
    m = jnp.max(y, axis=0, keepdims=True)
    z = y - m
    out_ref[...] = z - jnp.log(jnp.sum(jnp.exp(z), axis=0, keepdims=True))


def _combine(ss, i1, i2, p1, p2, w1, w2, k1, k2):
    return pl.pallas_call(
        _combine_body,
        out_shape=jax.ShapeDtypeStruct((T, 1), jnp.float32),
    )(ss, i1, i2, p1, p2, w1, w2, k1, k2)


def kernel(x, Wg, bg, W1, b1, W2, b2):
    xt = x.reshape(T, D)
    bg2 = bg.reshape(1, E)
    i1, i2, p1, p2, w1, w2, k1, k2, s1, s2 = _gate(xt, Wg, bg2)
    buf = _dispatch(xt, s1.reshape(T), s2.reshape(T))
    ss = _ffn(buf, W1, b1, W2, b2).reshape(E, CAPP)
    y = _combine(ss, i1, i2, p1, p2, w1, w2, k1, k2)
    return y.reshape(x.shape[0], x.shape[1])


# speedup vs baseline: 1.0082x; 1.0082x over previous
import functools

import jax
import jax.numpy as jnp
from jax import lax
from jax.experimental import pallas as pl
from jax.experimental.pallas import tpu as pltpu
from jax.experimental.pallas import tpu_sc as plsc

E = 64
K = 2
D = 768
H = 3072
T = 2048
CAP = 77
CAPP = 80
NENT = T * K
TRASH = E * CAPP
NSLOT = (E + 1) * CAPP
CH = 128
NCH = T // CH

_HI = jax.lax.Precision.HIGHEST


def _gate_body(x_ref, wg_ref, bg_ref,
               i1_ref, i2_ref, p1_ref, p2_ref, w1_ref, w2_ref,
               k1_ref, k2_ref, s1_ref, s2_ref,
               incl_s, tot_s):
    x = x_ref[...]
    logits = jnp.dot(x, wg_ref[...], preferred_element_type=jnp.float32)
    logits = logits + bg_ref[...]
    lane = lax.broadcasted_iota(jnp.int32, (T, E), 1)
    v1 = jnp.max(logits, axis=1, keepdims=True)
    i1 = jnp.min(jnp.where(logits == v1, lane, E), axis=1, keepdims=True)
    masked = jnp.where(lane == i1, -1e30, logits)
    v2 = jnp.max(masked, axis=1, keepdims=True)
    i2 = jnp.min(jnp.where(masked == v2, lane, E), axis=1, keepdims=True)
    g1 = 1.0 / (1.0 + jnp.exp(v2 - v1))
    g2 = 1.0 - g1

    oh1 = (lane == i1).astype(jnp.float32)
    oh2 = (lane == i2).astype(jnp.float32)
    ohc = oh1 + oh2

    r = lax.broadcasted_iota(jnp.int32, (CH, CH), 0)
    c = lax.broadcasted_iota(jnp.int32, (CH, CH), 1)
    tincl = (r >= c).astype(jnp.float32)
    for cc in range(NCH):
        blk = ohc[cc * CH:(cc + 1) * CH, :]
        inc = lax.dot(tincl, blk, precision=_HI)
        incl_s[cc * CH:(cc + 1) * CH, :] = inc
        tot_s[cc:cc + 1, :] = inc[CH - 1:CH, :]
    r16 = lax.broadcasted_iota(jnp.int32, (NCH, NCH), 0)
    c16 = lax.broadcasted_iota(jnp.int32, (NCH, NCH), 1)
    tstrict = (r16 > c16).astype(jnp.float32)
    offs = lax.dot(tstrict, tot_s[...], precision=_HI)

    i1_ref[...] = i1
    i2_ref[...] = i2
    for cc in range(NCH):
        sl = slice(cc * CH, (cc + 1) * CH)
        oh1c = oh1[sl, :]
        oh2c = oh2[sl, :]
        excl = incl_s[sl, :] - (oh1c + oh2c) + offs[cc:cc + 1, :]
        p1f = jnp.sum(oh1c * excl, axis=1, keepdims=True)
        p2f = jnp.sum(oh2c * (excl + oh1c), axis=1, keepdims=True)
        p1 = (p1f + 0.5).astype(jnp.int32)
        p2 = (p2f + 0.5).astype(jnp.int32)
        keep1 = p1 < CAP
        keep2 = p2 < CAP
        tglob = cc * CH + lax.broadcasted_iota(jnp.int32, (CH, 1), 0)
        went = tglob // (CH // K)
        i1c = i1[sl, :]
        i2c = i2[sl, :]
        p1_ref[sl, :] = jnp.minimum(p1, CAPP - 1)
        p2_ref[sl, :] = jnp.minimum(p2, CAPP - 1)
        w1_ref[sl, :] = g1[sl, :] * keep1.astype(jnp.float32)
        w2_ref[sl, :] = g2[sl, :] * keep2.astype(jnp.float32)
        k1_ref[sl, :] = keep1.astype(jnp.float32)
        k2_ref[sl, :] = keep2.astype(jnp.float32)
        s1_ref[sl, :] = jnp.where(keep1, i1c * CAPP + p1, TRASH + went)
        s2_ref[sl, :] = jnp.where(keep2, i2c * CAPP + p2, TRASH + went)


def _gate(xt, Wg, bg2):
    i32 = jnp.int32
    f32 = jnp.float32
    outs = [
        jax.ShapeDtypeStruct((T, 1), i32),
        jax.ShapeDtypeStruct((T, 1), i32),
        jax.ShapeDtypeStruct((T, 1), i32),
        jax.ShapeDtypeStruct((T, 1), i32),
        jax.ShapeDtypeStruct((T, 1), f32),
        jax.ShapeDtypeStruct((T, 1), f32),
        jax.ShapeDtypeStruct((T, 1), f32),
        jax.ShapeDtypeStruct((T, 1), f32),
        jax.ShapeDtypeStruct((T, 1), i32),
        jax.ShapeDtypeStruct((T, 1), i32),
    ]
    return pl.pallas_call(
        _gate_body,
        out_shape=outs,
        scratch_shapes=[
            pltpu.VMEM((T, E), f32),
            pltpu.VMEM((NCH, E), f32),
        ],
    )(xt, Wg, bg2)


_NC = 2
_NS = 16
_NW = _NC * _NS
_EPW = NENT // _NW


_TPW = T // _NW


def _dispatch_body(x_hbm, s1_hbm, s2_hbm, buf_hbm,
                   s1_v, s2_v, rows_v, sem, sem2):
    wid = lax.axis_index("s") * _NC + lax.axis_index("c")
    base = wid * _TPW
    cp = pltpu.async_copy(x_hbm.at[pl.ds(base, _TPW)], rows_v, sem)
    pltpu.sync_copy(s1_hbm.at[pl.ds(base, _TPW)], s1_v)
    pltpu.sync_copy(s2_hbm.at[pl.ds(base, _TPW)], s2_v)
    cp.wait()
    c1 = pltpu.async_copy(rows_v, buf_hbm.at[s1_v], sem)
    c2 = pltpu.async_copy(rows_v, buf_hbm.at[s2_v], sem2)
    c1.wait()
    c2.wait()


@functools.cache
def _dispatch_kernel():
    return pl.kernel(
        _dispatch_body,
        out_type=jax.ShapeDtypeStruct((NSLOT, D), jnp.float32),
        mesh=plsc.VectorSubcoreMesh(core_axis_name="c", subcore_axis_name="s"),
        scratch_types=[
            pltpu.VMEM((_TPW,), jnp.int32),
            pltpu.VMEM((_TPW,), jnp.int32),
            pltpu.VMEM((_TPW, D), jnp.float32),
            pltpu.SemaphoreType.DMA,
            pltpu.SemaphoreType.DMA,
        ],
    )


def _dispatch(xt, slot1, slot2):
    return _dispatch_kernel()(xt, slot1, slot2)


NHB = 2
HB = H // NHB


def _ffn_body(buf_ref, w1_ref, b1_ref, w2_ref, b2_ref, out_ref):
    hb = pl.program_id(1)
    xb = buf_ref[...]
    h = jnp.dot(xb, w1_ref[0], preferred_element_type=jnp.float32)
    h = jnp.maximum(h + b1_ref[0], 0.0)
    w2s = jnp.sum(w2_ref[0], axis=1, keepdims=True)
    val = jnp.dot(h, w2s, preferred_element_type=jnp.float32)

    @pl.when(hb == 0)
    def _():
        out_ref[0] = val + jnp.sum(b2_ref[...])

    @pl.when(hb != 0)
    def _():
        out_ref[0] += val

    @pl.when(hb == NHB - 1)
    def _():
        acc = out_ref[0]
        ok = (jnp.abs(acc) < 1e30) & (acc == acc)
        out_ref[0] = jnp.where(ok, acc, 0.0)


def _ffn(buf, W1, b1, W2, b2):
    return pl.pallas_call(
        _ffn_body,
        grid=(E, NHB),
        in_specs=[
            pl.BlockSpec((CAPP, D), lambda e, hb: (e, 0)),
            pl.BlockSpec((1, D, HB), lambda e, hb: (e, 0, hb)),
            pl.BlockSpec((1, 1, HB), lambda e, hb: (e, 0, hb)),
            pl.BlockSpec((1, HB, D), lambda e, hb: (e, hb, 0)),
            pl.BlockSpec((1, 1, D), lambda e, hb: (e, 0, 0)),
        ],
        out_specs=pl.BlockSpec((1, CAPP, 1), lambda e, hb: (e, 0, 0)),
        out_shape=jax.ShapeDtypeStruct((E, CAPP, 1), jnp.float32),
        compiler_params=pltpu.CompilerParams(
            dimension_semantics=("arbitrary", "arbitrary"),
        ),
    )(buf, W1, b1.reshape(E, 1, H), W2, b2.reshape(E, 1, D))


def _combine_body(ss_ref, i1_ref, i2_ref, p1_ref, p2_ref,
                  w1_ref, w2_ref, k1_ref, k2_ref, out_ref):
    ss = ss_ref[...]
    lane_e = lax.broadcasted_iota(jnp.int32, (T, E), 1)
    lane_c = lax.broadcasted_iota(jnp.int32, (T, CAPP), 1)

    oh1 = (lane_e == i1_ref[...]).astype(jnp.float32)
    a1 = lax.dot(oh1, ss, precision=_HI)
    sel1 = (lane_c == p1_ref[...]).astype(jnp.float32)
    val1 = jnp.sum(a1 * sel1, axis=1, keepdims=True)
    val1 = jnp.where(k1_ref[...] > 0.5, val1, 0.0)

    oh2 = (lane_e == i2_ref[...]).astype(jnp.float32)
    a2 = lax.dot(oh2, ss, precision=_HI)
    sel2 = (lane_c == p2_ref[...]).astype(jnp.float32)
    val2 = jnp.sum(a2 * sel2, axis=1, keepdims=True)
    val2 = jnp.where(k2_ref[...] > 0.5, val2, 0.0)

    y = w1_ref[...] * val1 + w2_ref[...] * val2
    m = jnp.max(y, axis=0, keepdims=True)
    z = y - m
    out_ref[...] = z - jnp.log(jnp.sum(jnp.exp(z), axis=0, keepdims=True))


def _combine(ss, i1, i2, p1, p2, w1, w2, k1, k2):
    return pl.pallas_call(
        _combine_body,
        out_shape=jax.ShapeDtypeStruct((T, 1), jnp.float32),
    )(ss, i1, i2, p1, p2, w1, w2, k1, k2)


def kernel(x, Wg, bg, W1, b1, W2, b2):
    xt = x.reshape(T, D)
    bg2 = bg.reshape(1, E)
    i1, i2, p1, p2, w1, w2, k1, k2, s1, s2 = _gate(xt, Wg, bg2)
    buf = _dispatch(xt, s1.reshape(T), s2.reshape(T))
    ss = _ffn(buf, W1, b1, W2, b2).reshape(E, CAPP)
    y = _combine(ss, i1, i2, p1, p2, w1, w2, k1, k2)
    return y.reshape(x.shape[0], x.shape[1])
